# initial kernel scaffold (unmeasured)
import jax
import jax.numpy as jnp
from jax import lax
from jax.experimental import pallas as pl
from jax.experimental.pallas import tpu as pltpu

N_DEV = 4
KT = 128


def kernel(Q, K, V):
    b, q, h, d = Q.shape
    kv = K.shape[1]
    n_steps = kv // KT
    scale = d ** -0.5

    def body(q_ref, k_ref, v_ref, out_ref,
             o_comm, l_comm, send_a, recv_a, send_b, recv_b):
        step = pl.program_id(0)

        @pl.when(step == 0)
        def _():
            o_comm[0] = jnp.zeros((b, h, d), jnp.float32)
            l_comm[0] = jnp.zeros((b, h), jnp.float32)

        qv = q_ref[:, 0, :, :]
        s = lax.dot_general(
            qv, k_ref[...],
            (((2,), (3,)), ((0, 1), (0, 2))),
            preferred_element_type=jnp.float32,
        )
        p = jnp.exp(s * scale)
        l_comm[0] = l_comm[0] + jnp.sum(p, axis=-1)
        o_comm[0] = o_comm[0] + lax.dot_general(
            p, v_ref[...],
            (((2,), (1,)), ((0, 1), (0, 2))),
            preferred_element_type=jnp.float32,
        )

        @pl.when(step == n_steps - 1)
        def _():
            my = lax.axis_index("i")
            pa = my ^ 1
            pb = 3 - my

            barrier = pltpu.get_barrier_semaphore()
            for nbr in (pa, pb):
                pl.semaphore_signal(
                    barrier, inc=1,
                    device_id=(nbr,), device_id_type=pl.DeviceIdType.MESH,
                )
            pl.semaphore_wait(barrier, 2)

            o_rdma_a = pltpu.make_async_remote_copy(
                src_ref=o_comm.at[0], dst_ref=o_comm.at[1],
                send_sem=send_a.at[0], recv_sem=recv_a.at[0],
                device_id=(pa,), device_id_type=pl.DeviceIdType.MESH,
            )
            l_rdma_a = pltpu.make_async_remote_copy(
                src_ref=l_comm.at[0], dst_ref=l_comm.at[1],
                send_sem=send_a.at[1], recv_sem=recv_a.at[1],
                device_id=(pa,), device_id_type=pl.DeviceIdType.MESH,
            )
            o_rdma_a.start()
            l_rdma_a.start()
            o_rdma_a.wait()
            l_rdma_a.wait()

            o_comm[2] = o_comm[0] + o_comm[1]
            l_comm[2] = l_comm[0] + l_comm[1]

            o_rdma_b = pltpu.make_async_remote_copy(
                src_ref=o_comm.at[2], dst_ref=o_comm.at[3],
                send_sem=send_b.at[0], recv_sem=recv_b.at[0],
                device_id=(pb,), device_id_type=pl.DeviceIdType.MESH,
            )
            l_rdma_b = pltpu.make_async_remote_copy(
                src_ref=l_comm.at[2], dst_ref=l_comm.at[3],
                send_sem=send_b.at[1], recv_sem=recv_b.at[1],
                device_id=(pb,), device_id_type=pl.DeviceIdType.MESH,
            )
            o_rdma_b.start()
            l_rdma_b.start()
            o_rdma_b.wait()
            l_rdma_b.wait()

            o_tot = o_comm[2] + o_comm[3]
            l_tot = l_comm[2] + l_comm[3]
            res = o_tot / l_tot[:, :, None]
            out_ref[...] = res.reshape(b, q, h, d)

    return pl.pallas_call(
        body,
        grid=(n_steps,),
        in_specs=[
            pl.BlockSpec((b, q, h, d), lambda i: (0, 0, 0, 0)),
            pl.BlockSpec((b, KT, h, d), lambda i: (0, i, 0, 0)),
            pl.BlockSpec((b, KT, h, d), lambda i: (0, i, 0, 0)),
        ],
        out_specs=pl.BlockSpec((b, q, h, d), lambda i: (0, 0, 0, 0)),
        out_shape=jax.ShapeDtypeStruct((b, q, h, d), jnp.float32),
        scratch_shapes=[
            pltpu.VMEM((4, b, h, d), jnp.float32),
            pltpu.VMEM((4, b, h), jnp.float32),
            pltpu.SemaphoreType.DMA((2,)),
            pltpu.SemaphoreType.DMA((2,)),
            pltpu.SemaphoreType.DMA((2,)),
            pltpu.SemaphoreType.DMA((2,)),
        ],
        compiler_params=pltpu.CompilerParams(
            dimension_semantics=("arbitrary",),
            collective_id=0,
        ),
    )(Q, K, V)


# baseline (device time: 368677 ns/iter reference)
import jax
import jax.numpy as jnp
from jax import lax
from jax.experimental import pallas as pl
from jax.experimental.pallas import tpu as pltpu

N_DEV = 4
KT = 64


def kernel(Q, K, V):
    b, q, h, d = Q.shape
    kv = K.shape[1]
    n_steps = kv // KT
    scale = d ** -0.5

    def body(q_ref, k_ref, v_ref, out_ref,
             o_comm, l_comm, send_a, recv_a, send_b, recv_b):
        step = pl.program_id(0)

        @pl.when(step == 0)
        def _():
            o_comm[0] = jnp.zeros((h, b, d), jnp.float32)
            l_comm[0] = jnp.zeros((h, b), jnp.float32)

        qs = q_ref[:, 0, :, :] * scale
        for hh in range(h):
            q_h = qs[:, hh, :]
            k_h = k_ref[:, :, hh, :]
            v_h = v_ref[:, :, hh, :]
            s_h = lax.dot_general(
                q_h, k_h,
                (((1,), (2,)), ((0,), (0,))),
                preferred_element_type=jnp.float32,
            )
            p_h = jnp.exp(s_h)
            l_comm[0, hh] = l_comm[0, hh] + jnp.sum(p_h, axis=-1)
            o_comm[0, hh] = o_comm[0, hh] + lax.dot_general(
                p_h, v_h,
                (((1,), (1,)), ((0,), (0,))),
                preferred_element_type=jnp.float32,
            )

        @pl.when(step == n_steps - 1)
        def _():
            my = lax.axis_index("i")
            pa = my ^ 1
            pb = 3 - my

            barrier = pltpu.get_barrier_semaphore()
            for nbr in (pa, pb):
                pl.semaphore_signal(
                    barrier, inc=1,
                    device_id=(nbr,), device_id_type=pl.DeviceIdType.MESH,
                )
            pl.semaphore_wait(barrier, 2)

            o_rdma_a = pltpu.make_async_remote_copy(
                src_ref=o_comm.at[0], dst_ref=o_comm.at[1],
                send_sem=send_a.at[0], recv_sem=recv_a.at[0],
                device_id=(pa,), device_id_type=pl.DeviceIdType.MESH,
            )
            l_rdma_a = pltpu.make_async_remote_copy(
                src_ref=l_comm.at[0], dst_ref=l_comm.at[1],
                send_sem=send_a.at[1], recv_sem=recv_a.at[1],
                device_id=(pa,), device_id_type=pl.DeviceIdType.MESH,
            )
            o_rdma_a.start()
            l_rdma_a.start()
            o_rdma_a.wait()
            l_rdma_a.wait()

            o_comm[2] = o_comm[0] + o_comm[1]
            l_comm[2] = l_comm[0] + l_comm[1]

            o_rdma_b = pltpu.make_async_remote_copy(
                src_ref=o_comm.at[2], dst_ref=o_comm.at[3],
                send_sem=send_b.at[0], recv_sem=recv_b.at[0],
                device_id=(pb,), device_id_type=pl.DeviceIdType.MESH,
            )
            l_rdma_b = pltpu.make_async_remote_copy(
                src_ref=l_comm.at[2], dst_ref=l_comm.at[3],
                send_sem=send_b.at[1], recv_sem=recv_b.at[1],
                device_id=(pb,), device_id_type=pl.DeviceIdType.MESH,
            )
            o_rdma_b.start()
            l_rdma_b.start()
            o_rdma_b.wait()
            l_rdma_b.wait()

            o_tot = o_comm[2] + o_comm[3]
            l_inv = 1.0 / (l_comm[2] + l_comm[3])
            for hh in range(h):
                out_ref[:, 0, hh, :] = o_tot[hh] * l_inv[hh][:, None]

    return pl.pallas_call(
        body,
        grid=(n_steps,),
        in_specs=[
            pl.BlockSpec((b, q, h, d), lambda i: (0, 0, 0, 0)),
            pl.BlockSpec((b, KT, h, d), lambda i: (0, i, 0, 0)),
            pl.BlockSpec((b, KT, h, d), lambda i: (0, i, 0, 0)),
        ],
        out_specs=pl.BlockSpec((b, q, h, d), lambda i: (0, 0, 0, 0)),
        out_shape=jax.ShapeDtypeStruct((b, q, h, d), jnp.float32),
        scratch_shapes=[
            pltpu.VMEM((4, h, b, d), jnp.float32),
            pltpu.VMEM((4, h, b), jnp.float32),
            pltpu.SemaphoreType.DMA((2,)),
            pltpu.SemaphoreType.DMA((2,)),
            pltpu.SemaphoreType.DMA((2,)),
            pltpu.SemaphoreType.DMA((2,)),
        ],
        compiler_params=pltpu.CompilerParams(
            dimension_semantics=("arbitrary",),
            collective_id=0,
            vmem_limit_bytes=48 * 1024 * 1024,
        ),
    )(Q, K, V)


# device time: 182418 ns/iter; 2.0211x vs baseline; 2.0211x over previous
import jax
import jax.numpy as jnp
from jax import lax
from jax.experimental import pallas as pl
from jax.experimental.pallas import tpu as pltpu

N_DEV = 4
KT = 128


def kernel(Q, K, V):
    b, q, h, d = Q.shape
    kv = K.shape[1]
    hd = h * d
    n_steps = kv // KT
    scale = d ** -0.5

    mask = (
        jnp.arange(hd, dtype=jnp.int32) // d
        == jnp.arange(h, dtype=jnp.int32)[:, None]
    ).astype(jnp.float32)
    qbd = mask[None, :, :] * (Q[:, 0].reshape(b, hd) * scale)[:, None, :]
    k2 = K.reshape(b, kv, hd)
    v2 = V.reshape(b, kv, hd)

    def body(qbd_ref, k_ref, v_ref, m_ref, out_ref,
             o_comm, l_comm, send_a, recv_a, send_b, recv_b):
        step = pl.program_id(0)

        @pl.when(step == 0)
        def _():
            o_comm[0] = jnp.zeros((b, hd), jnp.float32)
            l_comm[0] = jnp.zeros((b, h), jnp.float32)

        s = lax.dot_general(
            k_ref[...], qbd_ref[...],
            (((2,), (2,)), ((0,), (0,))),
            preferred_element_type=jnp.float32,
        )
        p = jnp.exp(s)
        l_comm[0] = l_comm[0] + jnp.sum(p, axis=1)
        p_wide = lax.dot_general(
            p, m_ref[...],
            (((2,), (0,)), ((), ())),
            preferred_element_type=jnp.float32,
        )
        o_comm[0] = o_comm[0] + jnp.sum(p_wide * v_ref[...], axis=1)

        @pl.when(step == n_steps - 1)
        def _():
            my = lax.axis_index("i")
            pa = my ^ 1
            pb = 3 - my

            barrier = pltpu.get_barrier_semaphore()
            for nbr in (pa, pb):
                pl.semaphore_signal(
                    barrier, inc=1,
                    device_id=(nbr,), device_id_type=pl.DeviceIdType.MESH,
                )
            pl.semaphore_wait(barrier, 2)

            o_rdma_a = pltpu.make_async_remote_copy(
                src_ref=o_comm.at[0], dst_ref=o_comm.at[1],
                send_sem=send_a.at[0], recv_sem=recv_a.at[0],
                device_id=(pa,), device_id_type=pl.DeviceIdType.MESH,
            )
            l_rdma_a = pltpu.make_async_remote_copy(
                src_ref=l_comm.at[0], dst_ref=l_comm.at[1],
                send_sem=send_a.at[1], recv_sem=recv_a.at[1],
                device_id=(pa,), device_id_type=pl.DeviceIdType.MESH,
            )
            o_rdma_a.start()
            l_rdma_a.start()
            o_rdma_a.wait()
            l_rdma_a.wait()

            o_comm[2] = o_comm[0] + o_comm[1]
            l_comm[2] = l_comm[0] + l_comm[1]

            o_rdma_b = pltpu.make_async_remote_copy(
                src_ref=o_comm.at[2], dst_ref=o_comm.at[3],
                send_sem=send_b.at[0], recv_sem=recv_b.at[0],
                device_id=(pb,), device_id_type=pl.DeviceIdType.MESH,
            )
            l_rdma_b = pltpu.make_async_remote_copy(
                src_ref=l_comm.at[2], dst_ref=l_comm.at[3],
                send_sem=send_b.at[1], recv_sem=recv_b.at[1],
                device_id=(pb,), device_id_type=pl.DeviceIdType.MESH,
            )
            o_rdma_b.start()
            l_rdma_b.start()
            o_rdma_b.wait()
            l_rdma_b.wait()

            o_tot = o_comm[2] + o_comm[3]
            l_tot = l_comm[2] + l_comm[3]
            l_wide = lax.dot_general(
                l_tot, m_ref[...],
                (((1,), (0,)), ((), ())),
                preferred_element_type=jnp.float32,
            )
            out_ref[...] = o_tot / l_wide

    out = pl.pallas_call(
        body,
        grid=(n_steps,),
        in_specs=[
            pl.BlockSpec((b, h, hd), lambda i: (0, 0, 0)),
            pl.BlockSpec((b, KT, hd), lambda i: (0, i, 0)),
            pl.BlockSpec((b, KT, hd), lambda i: (0, i, 0)),
            pl.BlockSpec((h, hd), lambda i: (0, 0)),
        ],
        out_specs=pl.BlockSpec((b, hd), lambda i: (0, 0)),
        out_shape=jax.ShapeDtypeStruct((b, hd), jnp.float32),
        scratch_shapes=[
            pltpu.VMEM((4, b, hd), jnp.float32),
            pltpu.VMEM((4, b, h), jnp.float32),
            pltpu.SemaphoreType.DMA((2,)),
            pltpu.SemaphoreType.DMA((2,)),
            pltpu.SemaphoreType.DMA((2,)),
            pltpu.SemaphoreType.DMA((2,)),
        ],
        compiler_params=pltpu.CompilerParams(
            dimension_semantics=("arbitrary",),
            collective_id=0,
            vmem_limit_bytes=48 * 1024 * 1024,
        ),
    )(qbd, k2, v2, mask)
    return out.reshape(b, q, h, d)
